# baseline (speedup 1.0000x reference)
import functools

import jax
import jax.numpy as jnp
from jax import lax
from jax.experimental import pallas as pl
from jax.experimental.pallas import tpu as pltpu
from jax.experimental.pallas import tpu_sc as plsc

N = 10000
E = 320000
D = 128
L = 16
W = 16
NSC = 2
NTILES = 16
WORKERS = NSC * NTILES
EPW = E // WORKERS
WINDOWS = EPW // W
NPAD = 10240
RPT = NPAD // NTILES
ZROWS = 16
NBUF = 3


def _self_ip_body(x_ref, s_ref):
    xb = x_ref[...]
    sq = xb * xb
    s_ref[...] = 2.0 * sq[:, D - 1] - jnp.sum(sq, axis=1)


def _final_body(p0_ref, p1_ref, x_ref, wn_ref, ws_ref, o_ref):
    P = p0_ref[...] + p1_ref[...]
    wsum = jnp.maximum(P[:, D - 1 : D], 1e-6)
    nf = P / wsum
    xb = x_ref[...]
    acc = jnp.dot(nf, wn_ref[...], preferred_element_type=jnp.float32)
    acc = acc + jnp.dot(xb, ws_ref[...], preferred_element_type=jnp.float32)
    o_ref[...] = jnp.maximum(acc, 0.0)


def _sc_edge_kernel(x_hbm, widx_hbm, s_hbm, out_hbm,
                    widx_all,
                    abrows, obuf, ssbuf, sidx,
                    semg, semsc, zbuf, accum):
    cid = lax.axis_index("c")
    sid = lax.axis_index("s")
    wid = sid * NSC + cid

    zeros = jnp.zeros((L,), jnp.float32)

    def zrow(r, carry):
        for k in range(D // L):
            zbuf[r, pl.ds(k * L, L)] = zeros
        return carry

    lax.fori_loop(0, ZROWS, zrow, 0)
    for j in range(RPT // ZROWS):
        pltpu.sync_copy(zbuf, accum.at[pl.ds(sid * RPT + j * ZROWS, ZROWS)])

    base = wid * EPW * 2
    pltpu.sync_copy(widx_hbm.at[pl.ds(base, EPW * 2)], widx_all)
    plsc.subcore_barrier()

    lane = lax.iota(jnp.int32, L)
    metric = jnp.where(lane == L - 1, -1.0, 1.0).astype(jnp.float32)
    is_last = lane == L - 1

    def issue_gathers(i, p):
        pltpu.async_copy(x_hbm.at[widx_all.at[pl.ds(i * 2 * W, 2 * W)]],
                         abrows[p], semg[p])
        pltpu.async_copy(s_hbm.at[widx_all.at[pl.ds(i * 2 * W, 2 * W)]],
                         ssbuf[p], semg[p])

    def drain_gathers(p):
        pltpu.make_async_copy(x_hbm.at[pl.ds(0, 2 * W)], abrows[p], semg[p]).wait()
        pltpu.make_async_copy(s_hbm.at[pl.ds(0, 2 * W)], ssbuf[p], semg[p]).wait()

    def drain_scatter(p):
        pltpu.make_async_copy(x_hbm.at[pl.ds(0, W)], obuf[p], semsc[p]).wait()

    for p in range(NBUF):
        issue_gathers(p, p)

    def process(i, p):
        drain_gathers(p)

        @pl.when(i >= NBUF)
        def _():
            drain_scatter(p)

        sidx[p][pl.ds(0, L)] = widx_all[pl.ds(i * 2 * W, L)]

        def qbody(q, qcarry):
            aav = ssbuf[p][pl.ds(0, L)]
            bbv = ssbuf[p][pl.ds(L, L)]
            denv = aav * bbv
            dinv = 1.0 / (jnp.maximum(jnp.abs(denv), 1e-9) * jnp.sign(denv))
            e0 = q * L
            for l in range(L):
                e = e0 + l
                a = [abrows[p][e, pl.ds(k * L, L)] for k in range(D // L)]
                b = [abrows[p][W + e, pl.ds(k * L, L)] for k in range(D // L)]
                t = a[0] * b[0]
                for k in range(1, D // L - 1):
                    t = t + a[k] * b[k]
                t = t + (a[D // L - 1] * b[D // L - 1]) * metric
                for k in (8, 4, 2, 1):
                    t = t + jnp.take(t, (lane + k) % L)
                wv = jnp.exp((denv[l] - t * t) * dinv[l])
                for k in range(D // L - 1):
                    obuf[p][e, pl.ds(k * L, L)] = wv * b[k]
                last = jnp.where(is_last, 1.0, b[D // L - 1])
                obuf[p][e, pl.ds((D // L - 1) * L, L)] = wv * last
            return qcarry

        lax.fori_loop(0, W // L, qbody, 0)
        pltpu.async_copy(obuf[p], accum.at[sidx[p]], semsc[p], add=True)

        @pl.when(i + NBUF < WINDOWS)
        def _():
            issue_gathers(i + NBUF, p)

    def window(i, carry):
        for p in range(NBUF):
            @pl.when(lax.rem(i, NBUF) == p)
            def _(p=p):
                process(i, p)

        return carry

    lax.fori_loop(0, WINDOWS, window, 0)
    for j in range(NBUF):
        drain_scatter((WINDOWS - NBUF + j) % NBUF)

    plsc.subcore_barrier()
    pltpu.sync_copy(accum.at[pl.ds(sid * RPT, RPT)],
                    out_hbm.at[cid, pl.ds(sid * RPT, RPT)])


_sc_edge = pl.kernel(
    _sc_edge_kernel,
    out_type=jax.ShapeDtypeStruct((NSC, NPAD, D), jnp.float32),
    mesh=plsc.VectorSubcoreMesh(
        core_axis_name="c", subcore_axis_name="s",
        num_cores=NSC, num_subcores=NTILES),
    scratch_types=[
        pltpu.VMEM((EPW * 2,), jnp.int32),
        [pltpu.VMEM((2 * W, D), jnp.float32)] * NBUF,
        [pltpu.VMEM((W, D), jnp.float32)] * NBUF,
        [pltpu.VMEM((2 * W,), jnp.float32)] * NBUF,
        [pltpu.VMEM((W,), jnp.int32)] * NBUF,
        [pltpu.SemaphoreType.DMA] * NBUF,
        [pltpu.SemaphoreType.DMA] * NBUF,
        pltpu.VMEM((ZROWS, D), jnp.float32),
        pltpu.VMEM_SHARED((NPAD, D), jnp.float32),
    ],
)


def kernel(x, edge_index, weight_neigh, weight_self):
    x = x.astype(jnp.float32)
    widx = edge_index.astype(jnp.int32).reshape(
        2, E // L, L).transpose(1, 0, 2).reshape(2 * E)

    BR = 1000
    s = pl.pallas_call(
        _self_ip_body,
        out_shape=jax.ShapeDtypeStruct((N,), jnp.float32),
    )(x)

    partials = _sc_edge(x, widx, s)

    wn_pad = jnp.zeros((D, D), jnp.float32).at[: D - 1, : D - 1].set(
        weight_neigh.T.astype(jnp.float32))
    ws_pad = jnp.zeros((D, D), jnp.float32).at[: D - 1, : D - 1].set(
        weight_self.T.astype(jnp.float32)).at[D - 1, D - 1].set(1.0)

    out = pl.pallas_call(
        _final_body,
        grid=(N // BR,),
        in_specs=[
            pl.BlockSpec((BR, D), lambda i: (i, 0)),
            pl.BlockSpec((BR, D), lambda i: (i, 0)),
            pl.BlockSpec((BR, D), lambda i: (i, 0)),
            pl.BlockSpec((D, D), lambda i: (0, 0)),
            pl.BlockSpec((D, D), lambda i: (0, 0)),
        ],
        out_specs=pl.BlockSpec((BR, D), lambda i: (i, 0)),
        out_shape=jax.ShapeDtypeStruct((N, D), jnp.float32),
    )(partials[0], partials[1], x, wn_pad, ws_pad)
    return out


# speedup vs baseline: 1.6003x; 1.6003x over previous
import functools

import jax
import jax.numpy as jnp
from jax import lax
from jax.experimental import pallas as pl
from jax.experimental.pallas import tpu as pltpu
from jax.experimental.pallas import tpu_sc as plsc

N = 10000
E = 320000
D = 128
L = 16
W = 16
NSC = 2
NTILES = 16
WORKERS = NSC * NTILES
EPW = E // WORKERS
WINDOWS = EPW // W
NPAD = 10240
RPT = NPAD // NTILES
ZROWS = 16
NBUF = 3


def _self_ip_body(x_ref, s_ref):
    xb = x_ref[...]
    sq = xb * xb
    s_ref[...] = 2.0 * sq[:, D - 1] - jnp.sum(sq, axis=1)


def _final_body(p0_ref, p1_ref, x_ref, wn_ref, ws_ref, o_ref):
    P = p0_ref[...] + p1_ref[...]
    wsum = jnp.maximum(P[:, D - 1 : D], 1e-6)
    nf = P / wsum
    xb = x_ref[...]
    acc = jnp.dot(nf, wn_ref[...], preferred_element_type=jnp.float32)
    acc = acc + jnp.dot(xb, ws_ref[...], preferred_element_type=jnp.float32)
    o_ref[...] = jnp.maximum(acc, 0.0)


def _sc_edge_kernel(x_hbm, row_hbm, col_hbm, s_hbm, out_hbm,
                    ridx_all, cidx_all,
                    arows, brows, obuf, aabuf, bbbuf, sidx,
                    semg, semsc, zbuf, accum):
    cid = lax.axis_index("c")
    sid = lax.axis_index("s")
    wid = sid * NSC + cid

    zeros = jnp.zeros((L,), jnp.float32)

    def zrow(r, carry):
        for k in range(D // L):
            zbuf[r, pl.ds(k * L, L)] = zeros
        return carry

    lax.fori_loop(0, ZROWS, zrow, 0)
    for j in range(RPT // ZROWS):
        pltpu.sync_copy(zbuf, accum.at[pl.ds(sid * RPT + j * ZROWS, ZROWS)])

    base = wid * EPW
    pltpu.sync_copy(row_hbm.at[pl.ds(base, EPW)], ridx_all)
    pltpu.sync_copy(col_hbm.at[pl.ds(base, EPW)], cidx_all)
    plsc.subcore_barrier()

    lane = lax.iota(jnp.int32, L)
    metric = jnp.where(lane == L - 1, -1.0, 1.0).astype(jnp.float32)
    is_last = lane == L - 1

    H = W // 2

    def issue_gathers(i, p):
        roff = i * W
        pltpu.async_copy(x_hbm.at[ridx_all.at[pl.ds(roff, H)]],
                         arows[p].at[pl.ds(0, H)], semg[p])
        pltpu.async_copy(x_hbm.at[ridx_all.at[pl.ds(roff + H, H)]],
                         arows[p].at[pl.ds(H, H)], semg[p])
        pltpu.async_copy(x_hbm.at[cidx_all.at[pl.ds(roff, H)]],
                         brows[p].at[pl.ds(0, H)], semg[p])
        pltpu.async_copy(x_hbm.at[cidx_all.at[pl.ds(roff + H, H)]],
                         brows[p].at[pl.ds(H, H)], semg[p])
        pltpu.async_copy(s_hbm.at[ridx_all.at[pl.ds(roff, W)]], aabuf[p], semg[p])
        pltpu.async_copy(s_hbm.at[cidx_all.at[pl.ds(roff, W)]], bbbuf[p], semg[p])

    def drain_gathers(p):
        pltpu.make_async_copy(x_hbm.at[pl.ds(0, W)], arows[p], semg[p]).wait()
        pltpu.make_async_copy(x_hbm.at[pl.ds(0, W)], brows[p], semg[p]).wait()
        pltpu.make_async_copy(s_hbm.at[pl.ds(0, W)], aabuf[p], semg[p]).wait()
        pltpu.make_async_copy(s_hbm.at[pl.ds(0, W)], bbbuf[p], semg[p]).wait()

    def drain_scatter(p):
        pltpu.make_async_copy(x_hbm.at[pl.ds(0, W)], obuf[p], semsc[p]).wait()

    for p in range(NBUF):
        issue_gathers(p, p)

    def process(i, p):
        drain_gathers(p)

        @pl.when(i >= NBUF)
        def _():
            drain_scatter(p)

        for q in range(W // L):
            sidx[p][pl.ds(q * L, L)] = ridx_all[pl.ds(i * W + q * L, L)]

        def qbody(q, qcarry):
            aav = aabuf[p][pl.ds(q * L, L)]
            bbv = bbbuf[p][pl.ds(q * L, L)]
            denv = aav * bbv
            dinv = 1.0 / (jnp.maximum(jnp.abs(denv), 1e-9) * jnp.sign(denv))
            e0 = q * L
            for l in range(L):
                e = e0 + l
                a = [arows[p][e, pl.ds(k * L, L)] for k in range(D // L)]
                b = [brows[p][e, pl.ds(k * L, L)] for k in range(D // L)]
                t = a[0] * b[0]
                for k in range(1, D // L - 1):
                    t = t + a[k] * b[k]
                t = t + (a[D // L - 1] * b[D // L - 1]) * metric
                for k in (8, 4, 2, 1):
                    t = t + jnp.take(t, (lane + k) % L)
                wv = jnp.exp((denv[l] - t * t) * dinv[l])
                for k in range(D // L - 1):
                    obuf[p][e, pl.ds(k * L, L)] = wv * b[k]
                last = jnp.where(is_last, 1.0, b[D // L - 1])
                obuf[p][e, pl.ds((D // L - 1) * L, L)] = wv * last
            return qcarry

        lax.fori_loop(0, W // L, qbody, 0)
        pltpu.async_copy(obuf[p], accum.at[sidx[p]], semsc[p], add=True)

        @pl.when(i + NBUF < WINDOWS)
        def _():
            issue_gathers(i + NBUF, p)

    def window(i, carry):
        for p in range(NBUF):
            @pl.when(lax.rem(i, NBUF) == p)
            def _(p=p):
                process(i, p)

        return carry

    lax.fori_loop(0, WINDOWS, window, 0)
    for j in range(NBUF):
        drain_scatter((WINDOWS - NBUF + j) % NBUF)

    plsc.subcore_barrier()
    pltpu.sync_copy(accum.at[pl.ds(sid * RPT, RPT)],
                    out_hbm.at[cid, pl.ds(sid * RPT, RPT)])


_sc_edge = pl.kernel(
    _sc_edge_kernel,
    out_type=jax.ShapeDtypeStruct((NSC, NPAD, D), jnp.float32),
    mesh=plsc.VectorSubcoreMesh(
        core_axis_name="c", subcore_axis_name="s",
        num_cores=NSC, num_subcores=NTILES),
    scratch_types=[
        pltpu.VMEM((EPW,), jnp.int32),
        pltpu.VMEM((EPW,), jnp.int32),
        [pltpu.VMEM((W, D), jnp.float32)] * NBUF,
        [pltpu.VMEM((W, D), jnp.float32)] * NBUF,
        [pltpu.VMEM((W, D), jnp.float32)] * NBUF,
        [pltpu.VMEM((W,), jnp.float32)] * NBUF,
        [pltpu.VMEM((W,), jnp.float32)] * NBUF,
        [pltpu.VMEM((W,), jnp.int32)] * NBUF,
        [pltpu.SemaphoreType.DMA] * NBUF,
        [pltpu.SemaphoreType.DMA] * NBUF,
        pltpu.VMEM((ZROWS, D), jnp.float32),
        pltpu.VMEM_SHARED((NPAD, D), jnp.float32),
    ],
)


def kernel(x, edge_index, weight_neigh, weight_self):
    x = x.astype(jnp.float32)
    row = edge_index[0].astype(jnp.int32)
    col = edge_index[1].astype(jnp.int32)

    BR = 1000
    s = pl.pallas_call(
        _self_ip_body,
        out_shape=jax.ShapeDtypeStruct((N,), jnp.float32),
    )(x)

    partials = _sc_edge(x, row, col, s)

    wn_pad = jnp.zeros((D, D), jnp.float32).at[: D - 1, : D - 1].set(
        weight_neigh.T.astype(jnp.float32))
    ws_pad = jnp.zeros((D, D), jnp.float32).at[: D - 1, : D - 1].set(
        weight_self.T.astype(jnp.float32)).at[D - 1, D - 1].set(1.0)

    out = pl.pallas_call(
        _final_body,
        grid=(N // BR,),
        in_specs=[
            pl.BlockSpec((BR, D), lambda i: (i, 0)),
            pl.BlockSpec((BR, D), lambda i: (i, 0)),
            pl.BlockSpec((BR, D), lambda i: (i, 0)),
            pl.BlockSpec((D, D), lambda i: (0, 0)),
            pl.BlockSpec((D, D), lambda i: (0, 0)),
        ],
        out_specs=pl.BlockSpec((BR, D), lambda i: (i, 0)),
        out_shape=jax.ShapeDtypeStruct((N, D), jnp.float32),
    )(partials[0], partials[1], x, wn_pad, ws_pad)
    return out
